# baseline (speedup 1.0000x reference)
import functools

import jax
import jax.numpy as jnp
from jax.experimental import pallas as pl
from jax.experimental.pallas import tpu as pltpu
from jax.experimental.pallas import tpu_sc as plsc

B, L, M, E = 16, 512, 2048, 256
F, K, NB = 256, 3, 256
_F32 = jnp.float32
_BF16 = jnp.bfloat16
_I32 = jnp.int32
_W = 128
_G = 4


def _shift_dn(x):
    return jnp.concatenate([jnp.zeros((1, x.shape[1]), x.dtype), x[:-1]], axis=0)


def _shift_up(x):
    return jnp.concatenate([x[1:], jnp.zeros((1, x.shape[1]), x.dtype)], axis=0)


def _conv3(xb, mask_dn, mask_up, w, bias):
    h = jnp.dot(xb, w[1], preferred_element_type=_F32)
    h = h + jnp.dot(_shift_dn(xb) * mask_dn, w[0], preferred_element_type=_F32)
    h = h + jnp.dot(_shift_up(xb) * mask_up, w[2], preferred_element_type=_F32)
    return h + bias[None, :]


def _ln_bf16(h, s, b):
    mu = jnp.mean(h, axis=-1, keepdims=True)
    var = jnp.mean((h - mu) * (h - mu), axis=-1, keepdims=True)
    rs = 1.0 / jnp.sqrt(var + 1e-5)
    return ((h - mu) * rs * s[None, :] + b[None, :]).astype(_BF16)


def _predictor(xb, mask_dn, mask_up, c1w, c1b, ln1s, ln1b, c2w, c2b,
               ln2s, ln2b, lw, lb):
    h = jax.nn.relu(_conv3(xb, mask_dn, mask_up, c1w, c1b))
    h = _ln_bf16(h, ln1s, ln1b)
    h = jax.nn.relu(_conv3(h, mask_dn, mask_up, c2w, c2b))
    mu = jnp.mean(h, axis=-1, keepdims=True)
    var = jnp.mean((h - mu) * (h - mu), axis=-1, keepdims=True)
    rs = 1.0 / jnp.sqrt(var + 1e-5)
    w2 = ln2s * lw
    c2 = jnp.sum(ln2b * lw) + lb
    return jnp.sum((h - mu) * rs * w2[None, :], axis=1) + c2


def _bucket_emb(target, bins, emb_b):
    c = (bins[None, :] < target[:, None]).astype(_BF16)
    ones_col = jnp.ones((L, 1), _BF16)
    c_prev = jnp.concatenate([ones_col, c[:, :-1]], axis=1)
    not_c = jnp.concatenate([1.0 - c[:, :-1], ones_col], axis=1)
    return jnp.dot(c_prev * not_c, emb_b, preferred_element_type=_F32)


def _emb_body(x_ref, pt_ref, et_ref, dur_ref, pbins, ebins, pemb, eemb,
              x1b_ref, x2_ref, gidx_ref):
    b = pl.program_id(0)
    x0 = x_ref[0]
    p_emb = _bucket_emb(pt_ref[b, :], pbins[0], pemb[...].astype(_BF16))
    e_emb = _bucket_emb(et_ref[b, :], ebins[0], eemb[...].astype(_BF16))
    x1 = x0 + p_emb
    x1b_ref[0] = x1.astype(_BF16)
    x2_ref[0] = x1 + e_emb

    df = dur_ref[b, :].astype(_BF16)[None, :]
    tri = (jax.lax.broadcasted_iota(_I32, (L, L), 0)
           < jax.lax.broadcasted_iota(_I32, (L, L), 1)).astype(_BF16)
    excl = jnp.dot(df, tri, preferred_element_type=_F32)
    miota = jax.lax.broadcasted_iota(_I32, (M, 1), 0).astype(_F32)
    cmp = (excl <= miota).astype(_BF16)
    cnt = jnp.dot(cmp, jnp.ones((L, 1), _BF16),
                  preferred_element_type=_F32)
    gidx_ref[0, 0, :] = cnt[:, 0].astype(_I32) - 1 + b * L


def _pred_body(x0_ref, x1b_ref,
               dw1, db1, ds1, dbb1, dw2, db2, ds2, dbb2, dlw, dlb,
               pw1, pb1, ps1, pbb1, pw2, pb2, ps2, pbb2, plw, plb,
               ew1, eb1, es1, ebb1, ew2, eb2, es2, ebb2, elw, elb,
               logd_ref, pitch_ref, energy_ref):
    wb = lambda w: w[...].astype(_BF16)
    x0b = x0_ref[...].reshape(_G * L, E).astype(_BF16)
    x1b = x1b_ref[...].reshape(_G * L, E)
    riota = jax.lax.broadcasted_iota(_I32, (_G * L, 1), 0)
    mask_dn = ((riota & (L - 1)) != 0).astype(_BF16)
    mask_up = ((riota & (L - 1)) != (L - 1)).astype(_BF16)
    logd_ref[...] = _predictor(
        x0b, mask_dn, mask_up, wb(dw1), db1[0], ds1[0], dbb1[0], wb(dw2),
        db2[0], ds2[0], dbb2[0], dlw[0], dlb[0, 0]).reshape(_G, 1, L)
    pitch_ref[...] = _predictor(
        x0b, mask_dn, mask_up, wb(pw1), pb1[0], ps1[0], pbb1[0], wb(pw2),
        pb2[0], ps2[0], pbb2[0], plw[0], plb[0, 0]).reshape(_G, 1, L)
    energy_ref[...] = _predictor(
        x1b, mask_dn, mask_up, wb(ew1), eb1[0], es1[0], ebb1[0], wb(ew2),
        eb2[0], es2[0], ebb2[0], elw[0], elb[0, 0]).reshape(_G, 1, L)


def _const(*shape):
    nd = len(shape)
    return pl.BlockSpec(shape, lambda b, _n=nd: (0,) * _n)


def _sc_gather(x2_flat, gidx_flat):
    mesh = plsc.VectorSubcoreMesh(core_axis_name='c', subcore_axis_name='s')
    nc, ns = 2, 16
    b_per_w = (B * M) // (nc * ns)
    nch = b_per_w // _W

    @functools.partial(
        pl.kernel, mesh=mesh,
        out_type=jax.ShapeDtypeStruct((B * M, E), _F32),
        scratch_types=[
            pltpu.VMEM((_W,), _I32), pltpu.VMEM((_W,), _I32),
            pltpu.VMEM((_W, E), _F32), pltpu.VMEM((_W, E), _F32),
            pltpu.SemaphoreType.DMA, pltpu.SemaphoreType.DMA,
        ],
    )
    def k(x_hbm, idx_hbm, out_hbm, idx0, idx1, rows0, rows1, sem0, sem1):
        wid = jax.lax.axis_index('s') * nc + jax.lax.axis_index('c')
        base0 = wid * b_per_w

        def issue(c, idx_v, rows_v, sem):
            pltpu.sync_copy(idx_hbm.at[pl.ds(base0 + c * _W, _W)], idx_v)
            pltpu.async_copy(x_hbm.at[idx_v], rows_v, sem)

        def drain(c, idx_v, rows_v, sem):
            pltpu.make_async_copy(x_hbm.at[idx_v], rows_v, sem).wait()
            pltpu.sync_copy(rows_v, out_hbm.at[pl.ds(base0 + c * _W, _W)])

        issue(0, idx0, rows0, sem0)

        @pl.loop(0, nch // 2)
        def _(j):
            c0 = 2 * j
            issue(c0 + 1, idx1, rows1, sem1)
            drain(c0, idx0, rows0, sem0)

            @pl.when(c0 + 2 < nch)
            def _():
                issue(c0 + 2, idx0, rows0, sem0)

            drain(c0 + 1, idx1, rows1, sem1)

    return k(x2_flat, gidx_flat)


def kernel(hidden_phoneme_sequence, sequence_mask, frame_masks, pitch_target,
           energy_target, duration_target, duration_scale, pitch_scale,
           energy_scale,
           dur_c1w, dur_c1b, dur_ln1s, dur_ln1b, dur_c2w, dur_c2b,
           dur_ln2s, dur_ln2b, dur_lw, dur_lb,
           pit_c1w, pit_c1b, pit_ln1s, pit_ln1b, pit_c2w, pit_c2b,
           pit_ln2s, pit_ln2b, pit_lw, pit_lb,
           ene_c1w, ene_c1b, ene_ln1s, ene_ln1b, ene_c2w, ene_c2b,
           ene_ln2s, ene_ln2b, ene_lw, ene_lb,
           pitch_bins, energy_bins, pitch_emb, energy_emb):
    x0 = hidden_phoneme_sequence
    r2 = lambda a: a.reshape(1, -1)

    x1b, x2, gidx = pl.pallas_call(
        _emb_body,
        grid=(B,),
        in_specs=[
            pl.BlockSpec((1, L, E), lambda b: (b, 0, 0)),
            _const(B, L), _const(B, L), _const(B, L),
            _const(1, NB), _const(1, NB), _const(NB, E), _const(NB, E),
        ],
        out_specs=(pl.BlockSpec((1, L, E), lambda b: (b, 0, 0)),
                   pl.BlockSpec((1, L, E), lambda b: (b, 0, 0)),
                   pl.BlockSpec((1, 1, M), lambda b: (b, 0, 0))),
        out_shape=(jax.ShapeDtypeStruct((B, L, E), _BF16),
                   jax.ShapeDtypeStruct((B, L, E), _F32),
                   jax.ShapeDtypeStruct((B, 1, M), _I32)),
        compiler_params=pltpu.CompilerParams(
            dimension_semantics=("parallel",)),
    )(x0, pitch_target, energy_target, duration_target.astype(_I32),
      r2(pitch_bins), r2(energy_bins), pitch_emb, energy_emb)

    xout = _sc_gather(x2.reshape(B * L, E), gidx.reshape(B * M))

    wts = []
    w_specs = []
    for t in ((dur_c1w, dur_c1b, dur_ln1s, dur_ln1b, dur_c2w, dur_c2b,
               dur_ln2s, dur_ln2b, dur_lw, dur_lb),
              (pit_c1w, pit_c1b, pit_ln1s, pit_ln1b, pit_c2w, pit_c2b,
               pit_ln2s, pit_ln2b, pit_lw, pit_lb),
              (ene_c1w, ene_c1b, ene_ln1s, ene_ln1b, ene_c2w, ene_c2b,
               ene_ln2s, ene_ln2b, ene_lw, ene_lb)):
        c1w, c1b, ln1s, ln1b, c2w, c2b, ln2s, ln2b, lw, lb = t
        wts += [c1w, r2(c1b), r2(ln1s), r2(ln1b), c2w, r2(c2b), r2(ln2s),
                r2(ln2b), lw.reshape(1, F), lb.reshape(1, 1)]
        w_specs += [
            _const(K, E, F), _const(1, F), _const(1, F), _const(1, F),
            _const(K, F, F), _const(1, F), _const(1, F), _const(1, F),
            _const(1, F), _const(1, 1),
        ]

    logd, pitch, energy = pl.pallas_call(
        _pred_body,
        grid=(B // _G,),
        in_specs=[pl.BlockSpec((_G, L, E), lambda b: (b, 0, 0)),
                  pl.BlockSpec((_G, L, E), lambda b: (b, 0, 0)),
                  *w_specs],
        out_specs=(pl.BlockSpec((_G, 1, L), lambda b: (b, 0, 0)),) * 3,
        out_shape=(jax.ShapeDtypeStruct((B, 1, L), _F32),) * 3,
        compiler_params=pltpu.CompilerParams(
            dimension_semantics=("parallel",)),
    )(x0, x1b, *wts)

    return (logd.reshape(B, L), pitch.reshape(B, L), energy.reshape(B, L),
            xout.reshape(B, M, E), frame_masks)


# speedup vs baseline: 1.1615x; 1.1615x over previous
import functools

import jax
import jax.numpy as jnp
from jax.experimental import pallas as pl
from jax.experimental.pallas import tpu as pltpu
from jax.experimental.pallas import tpu_sc as plsc

B, L, M, E = 16, 512, 2048, 256
F, K, NB = 256, 3, 256
_F32 = jnp.float32
_BF16 = jnp.bfloat16
_I32 = jnp.int32
_W = 128
_G = 2


def _shift_dn(x):
    return jnp.concatenate([jnp.zeros((1, x.shape[1]), x.dtype), x[:-1]], axis=0)


def _shift_up(x):
    return jnp.concatenate([x[1:], jnp.zeros((1, x.shape[1]), x.dtype)], axis=0)


def _conv3(xb, mask_dn, mask_up, w, bias):
    h = jnp.dot(xb, w[1], preferred_element_type=_F32)
    h = h + jnp.dot(_shift_dn(xb) * mask_dn, w[0], preferred_element_type=_F32)
    h = h + jnp.dot(_shift_up(xb) * mask_up, w[2], preferred_element_type=_F32)
    return h + bias[None, :]


def _ln_bf16(h, s, b):
    mu = jnp.mean(h, axis=-1, keepdims=True)
    var = jnp.mean((h - mu) * (h - mu), axis=-1, keepdims=True)
    rs = 1.0 / jnp.sqrt(var + 1e-5)
    return ((h - mu) * rs * s[None, :] + b[None, :]).astype(_BF16)


def _predictor(xb, mask_dn, mask_up, c1w, c1b, ln1s, ln1b, c2w, c2b,
               ln2s, ln2b, lw, lb):
    h = jax.nn.relu(_conv3(xb, mask_dn, mask_up, c1w, c1b))
    h = _ln_bf16(h, ln1s, ln1b)
    h = jax.nn.relu(_conv3(h, mask_dn, mask_up, c2w, c2b))
    mu = jnp.mean(h, axis=-1, keepdims=True)
    var = jnp.mean((h - mu) * (h - mu), axis=-1, keepdims=True)
    rs = 1.0 / jnp.sqrt(var + 1e-5)
    hn = ((h - mu) * rs).astype(_BF16)
    w2 = (ln2s * lw)[None, :].astype(_BF16)
    c2 = jnp.sum(ln2b * lw) + lb
    res = jax.lax.dot_general(w2, hn, (((1,), (1,)), ((), ())),
                              preferred_element_type=_F32)
    return res[0] + c2


def _bucket_emb(target, bins, emb_b):
    c = (bins[None, :] < target[:, None]).astype(_BF16)
    ones_col = jnp.ones((L, 1), _BF16)
    c_prev = jnp.concatenate([ones_col, c[:, :-1]], axis=1)
    not_c = jnp.concatenate([1.0 - c[:, :-1], ones_col], axis=1)
    return jnp.dot(c_prev * not_c, emb_b, preferred_element_type=_F32)


def _emb_body(x_ref, pt_ref, et_ref, dur_ref, pbins, ebins, pemb, eemb,
              x0b_ref, x1b_ref, x2_ref, gidx_ref):
    b = pl.program_id(0)
    x0 = x_ref[0]
    x0b_ref[0] = x0.astype(_BF16)
    p_emb = _bucket_emb(pt_ref[b, :], pbins[0], pemb[...].astype(_BF16))
    e_emb = _bucket_emb(et_ref[b, :], ebins[0], eemb[...].astype(_BF16))
    x1 = x0 + p_emb
    x1b_ref[0] = x1.astype(_BF16)
    x2_ref[0] = x1 + e_emb

    df = dur_ref[b, :].astype(_BF16)[None, :]
    tri = (jax.lax.broadcasted_iota(_I32, (L, L), 0)
           < jax.lax.broadcasted_iota(_I32, (L, L), 1)).astype(_BF16)
    excl = jnp.dot(df, tri, preferred_element_type=_F32)
    miota = jax.lax.broadcasted_iota(_I32, (M, 1), 0).astype(_F32)
    cmp = (excl <= miota).astype(_BF16)
    cnt = jnp.dot(cmp, jnp.ones((L, 1), _BF16),
                  preferred_element_type=_F32)
    gidx_ref[0, 0, :] = cnt[:, 0].astype(_I32) - 1 + b * L


def _pred_body(x0b_ref, x1b_ref,
               dw1, db1, ds1, dbb1, dw2, db2, ds2, dbb2, dlw, dlb,
               pw1, pb1, ps1, pbb1, pw2, pb2, ps2, pbb2, plw, plb,
               ew1, eb1, es1, ebb1, ew2, eb2, es2, ebb2, elw, elb,
               logd_ref, pitch_ref, energy_ref):
    wb = lambda w: w[...].astype(_BF16)
    x0b = x0b_ref[...].reshape(_G * L, E)
    x1b = x1b_ref[...].reshape(_G * L, E)
    riota = jax.lax.broadcasted_iota(_I32, (_G * L, 1), 0)
    mask_dn = ((riota & (L - 1)) != 0).astype(_BF16)
    mask_up = ((riota & (L - 1)) != (L - 1)).astype(_BF16)
    logd_ref[0, 0, :] = _predictor(
        x0b, mask_dn, mask_up, wb(dw1), db1[0], ds1[0], dbb1[0], wb(dw2),
        db2[0], ds2[0], dbb2[0], dlw[0], dlb[0, 0])
    pitch_ref[0, 0, :] = _predictor(
        x0b, mask_dn, mask_up, wb(pw1), pb1[0], ps1[0], pbb1[0], wb(pw2),
        pb2[0], ps2[0], pbb2[0], plw[0], plb[0, 0])
    energy_ref[0, 0, :] = _predictor(
        x1b, mask_dn, mask_up, wb(ew1), eb1[0], es1[0], ebb1[0], wb(ew2),
        eb2[0], es2[0], ebb2[0], elw[0], elb[0, 0])


def _const(*shape):
    nd = len(shape)
    return pl.BlockSpec(shape, lambda b, _n=nd: (0,) * _n)


def _sc_gather(x2_flat, gidx_flat):
    mesh = plsc.VectorSubcoreMesh(core_axis_name='c', subcore_axis_name='s')
    nc, ns = 2, 16
    b_per_w = (B * M) // (nc * ns)
    nch = b_per_w // _W

    @functools.partial(
        pl.kernel, mesh=mesh,
        out_type=jax.ShapeDtypeStruct((B * M, E), _F32),
        scratch_types=[
            pltpu.VMEM((_W,), _I32), pltpu.VMEM((_W,), _I32),
            pltpu.VMEM((_W, E), _F32), pltpu.VMEM((_W, E), _F32),
            pltpu.SemaphoreType.DMA, pltpu.SemaphoreType.DMA,
        ],
    )
    def k(x_hbm, idx_hbm, out_hbm, idx0, idx1, rows0, rows1, sem0, sem1):
        wid = jax.lax.axis_index('s') * nc + jax.lax.axis_index('c')
        base0 = wid * b_per_w

        def issue(c, idx_v, rows_v, sem):
            pltpu.sync_copy(idx_hbm.at[pl.ds(base0 + c * _W, _W)], idx_v)
            pltpu.async_copy(x_hbm.at[idx_v], rows_v, sem)

        def drain(c, idx_v, rows_v, sem):
            pltpu.make_async_copy(x_hbm.at[idx_v], rows_v, sem).wait()
            pltpu.sync_copy(rows_v, out_hbm.at[pl.ds(base0 + c * _W, _W)])

        issue(0, idx0, rows0, sem0)

        @pl.loop(0, nch // 2)
        def _(j):
            c0 = 2 * j
            issue(c0 + 1, idx1, rows1, sem1)
            drain(c0, idx0, rows0, sem0)

            @pl.when(c0 + 2 < nch)
            def _():
                issue(c0 + 2, idx0, rows0, sem0)

            drain(c0 + 1, idx1, rows1, sem1)

    return k(x2_flat, gidx_flat)


def kernel(hidden_phoneme_sequence, sequence_mask, frame_masks, pitch_target,
           energy_target, duration_target, duration_scale, pitch_scale,
           energy_scale,
           dur_c1w, dur_c1b, dur_ln1s, dur_ln1b, dur_c2w, dur_c2b,
           dur_ln2s, dur_ln2b, dur_lw, dur_lb,
           pit_c1w, pit_c1b, pit_ln1s, pit_ln1b, pit_c2w, pit_c2b,
           pit_ln2s, pit_ln2b, pit_lw, pit_lb,
           ene_c1w, ene_c1b, ene_ln1s, ene_ln1b, ene_c2w, ene_c2b,
           ene_ln2s, ene_ln2b, ene_lw, ene_lb,
           pitch_bins, energy_bins, pitch_emb, energy_emb):
    x0 = hidden_phoneme_sequence
    r2 = lambda a: a.reshape(1, -1)

    x0b, x1b, x2, gidx = pl.pallas_call(
        _emb_body,
        grid=(B,),
        in_specs=[
            pl.BlockSpec((1, L, E), lambda b: (b, 0, 0)),
            _const(B, L), _const(B, L), _const(B, L),
            _const(1, NB), _const(1, NB), _const(NB, E), _const(NB, E),
        ],
        out_specs=(pl.BlockSpec((1, L, E), lambda b: (b, 0, 0)),
                   pl.BlockSpec((1, L, E), lambda b: (b, 0, 0)),
                   pl.BlockSpec((1, L, E), lambda b: (b, 0, 0)),
                   pl.BlockSpec((1, 1, M), lambda b: (b, 0, 0))),
        out_shape=(jax.ShapeDtypeStruct((B, L, E), _BF16),
                   jax.ShapeDtypeStruct((B, L, E), _BF16),
                   jax.ShapeDtypeStruct((B, L, E), _F32),
                   jax.ShapeDtypeStruct((B, 1, M), _I32)),
    )(x0, pitch_target, energy_target, duration_target.astype(_I32),
      r2(pitch_bins), r2(energy_bins), pitch_emb, energy_emb)

    xout = _sc_gather(x2.reshape(B * L, E), gidx.reshape(B * M))

    wts = []
    w_specs = []
    for t in ((dur_c1w, dur_c1b, dur_ln1s, dur_ln1b, dur_c2w, dur_c2b,
               dur_ln2s, dur_ln2b, dur_lw, dur_lb),
              (pit_c1w, pit_c1b, pit_ln1s, pit_ln1b, pit_c2w, pit_c2b,
               pit_ln2s, pit_ln2b, pit_lw, pit_lb),
              (ene_c1w, ene_c1b, ene_ln1s, ene_ln1b, ene_c2w, ene_c2b,
               ene_ln2s, ene_ln2b, ene_lw, ene_lb)):
        c1w, c1b, ln1s, ln1b, c2w, c2b, ln2s, ln2b, lw, lb = t
        wts += [c1w, r2(c1b), r2(ln1s), r2(ln1b), c2w, r2(c2b), r2(ln2s),
                r2(ln2b), lw.reshape(1, F), lb.reshape(1, 1)]
        w_specs += [
            _const(K, E, F), _const(1, F), _const(1, F), _const(1, F),
            _const(K, F, F), _const(1, F), _const(1, F), _const(1, F),
            _const(1, F), _const(1, 1),
        ]

    logd, pitch, energy = pl.pallas_call(
        _pred_body,
        grid=(B // _G,),
        in_specs=[pl.BlockSpec((_G, L, E), lambda b: (b, 0, 0)),
                  pl.BlockSpec((_G, L, E), lambda b: (b, 0, 0)),
                  *w_specs],
        out_specs=(pl.BlockSpec((1, 1, _G * L), lambda b: (b, 0, 0)),) * 3,
        out_shape=(jax.ShapeDtypeStruct((B // _G, 1, _G * L), _F32),) * 3,
    )(x0b, x1b, *wts)

    return (logd.reshape(B, L), pitch.reshape(B, L), energy.reshape(B, L),
            xout.reshape(B, M, E), frame_masks)


# speedup vs baseline: 1.2759x; 1.0984x over previous
import functools

import jax
import jax.numpy as jnp
from jax.experimental import pallas as pl
from jax.experimental.pallas import tpu as pltpu
from jax.experimental.pallas import tpu_sc as plsc

B, L, M, E = 16, 512, 2048, 256
F, K, NB = 256, 3, 256
_F32 = jnp.float32
_BF16 = jnp.bfloat16
_I32 = jnp.int32
_W = 128
_G = 2


def _shift_dn(x):
    return jnp.concatenate([jnp.zeros((1, x.shape[1]), x.dtype), x[:-1]], axis=0)


def _shift_up(x):
    return jnp.concatenate([x[1:], jnp.zeros((1, x.shape[1]), x.dtype)], axis=0)


def _conv3(xb, mask_dn, mask_up, w, bias):
    h = jnp.dot(xb, w[1], preferred_element_type=_F32)
    h = h + jnp.dot(_shift_dn(xb) * mask_dn, w[0], preferred_element_type=_F32)
    h = h + jnp.dot(_shift_up(xb) * mask_up, w[2], preferred_element_type=_F32)
    return h + bias[None, :]


def _ln_bf16(h, s, b):
    mu = jnp.mean(h, axis=-1, keepdims=True)
    var = jnp.mean((h - mu) * (h - mu), axis=-1, keepdims=True)
    rs = 1.0 / jnp.sqrt(var + 1e-5)
    return ((h - mu) * rs * s[None, :] + b[None, :]).astype(_BF16)


def _predictor(xb, mask_dn, mask_up, c1w, c1b, ln1s, ln1b, c2w, c2b,
               ln2s, ln2b, lw, lb):
    h = jax.nn.relu(_conv3(xb, mask_dn, mask_up, c1w, c1b))
    h = _ln_bf16(h, ln1s, ln1b)
    h = jax.nn.relu(_conv3(h, mask_dn, mask_up, c2w, c2b))
    mu = jnp.mean(h, axis=-1, keepdims=True)
    var = jnp.mean((h - mu) * (h - mu), axis=-1, keepdims=True)
    rs = 1.0 / jnp.sqrt(var + 1e-5)
    hn = ((h - mu) * rs).astype(_BF16)
    w2 = (ln2s * lw)[None, :].astype(_BF16)
    c2 = jnp.sum(ln2b * lw) + lb
    res = jax.lax.dot_general(w2, hn, (((1,), (1,)), ((), ())),
                              preferred_element_type=_F32)
    return res[0] + c2


def _bucket_emb(target, bins, emb_b):
    c = (bins[None, :] < target[:, None]).astype(_BF16)
    ones_col = jnp.ones((L, 1), _BF16)
    c_prev = jnp.concatenate([ones_col, c[:, :-1]], axis=1)
    not_c = jnp.concatenate([1.0 - c[:, :-1], ones_col], axis=1)
    return jnp.dot(c_prev * not_c, emb_b, preferred_element_type=_F32)


def _emb_body(x_ref, pt_ref, et_ref, dur_ref, pbins, ebins, pemb, eemb,
              x0b_ref, x1b_ref, x2_ref, gidx_ref):
    b = pl.program_id(0)
    x0 = x_ref[0]
    x0b_ref[0] = x0.astype(_BF16)
    p_emb = _bucket_emb(pt_ref[b, :], pbins[0], pemb[...].astype(_BF16))
    e_emb = _bucket_emb(et_ref[b, :], ebins[0], eemb[...].astype(_BF16))
    x1 = x0 + p_emb
    x1b_ref[0] = x1.astype(_BF16)
    x2_ref[0] = x1 + e_emb

    df = dur_ref[b, :].astype(_BF16)[None, :]
    tri = (jax.lax.broadcasted_iota(_I32, (L, L), 0)
           < jax.lax.broadcasted_iota(_I32, (L, L), 1)).astype(_BF16)
    excl = jnp.dot(df, tri, preferred_element_type=_F32)
    excl_col = excl.reshape(L, 1)
    miota = jax.lax.broadcasted_iota(_I32, (1, M), 1).astype(_F32)
    cmp_t = (excl_col <= miota).astype(_BF16)
    cnt = jnp.dot(jnp.ones((1, L), _BF16), cmp_t,
                  preferred_element_type=_F32)
    gidx_ref[0, 0, :] = cnt[0].astype(_I32) - 1 + b * L


def _pred_body(x0b_ref, x1b_ref,
               dw1, db1, ds1, dbb1, dw2, db2, ds2, dbb2, dlw, dlb,
               pw1, pb1, ps1, pbb1, pw2, pb2, ps2, pbb2, plw, plb,
               ew1, eb1, es1, ebb1, ew2, eb2, es2, ebb2, elw, elb,
               logd_ref, pitch_ref, energy_ref):
    wb = lambda w: w[...].astype(_BF16)
    x0b = x0b_ref[...].reshape(_G * L, E)
    x1b = x1b_ref[...].reshape(_G * L, E)
    riota = jax.lax.broadcasted_iota(_I32, (_G * L, 1), 0)
    mask_dn = ((riota & (L - 1)) != 0).astype(_BF16)
    mask_up = ((riota & (L - 1)) != (L - 1)).astype(_BF16)
    logd_ref[0, 0, :] = _predictor(
        x0b, mask_dn, mask_up, wb(dw1), db1[0], ds1[0], dbb1[0], wb(dw2),
        db2[0], ds2[0], dbb2[0], dlw[0], dlb[0, 0])
    pitch_ref[0, 0, :] = _predictor(
        x0b, mask_dn, mask_up, wb(pw1), pb1[0], ps1[0], pbb1[0], wb(pw2),
        pb2[0], ps2[0], pbb2[0], plw[0], plb[0, 0])
    energy_ref[0, 0, :] = _predictor(
        x1b, mask_dn, mask_up, wb(ew1), eb1[0], es1[0], ebb1[0], wb(ew2),
        eb2[0], es2[0], ebb2[0], elw[0], elb[0, 0])


def _const(*shape):
    nd = len(shape)
    return pl.BlockSpec(shape, lambda b, _n=nd: (0,) * _n)


def _sc_gather(x2_flat, gidx_flat):
    mesh = plsc.VectorSubcoreMesh(core_axis_name='c', subcore_axis_name='s')
    nc, ns = 2, 16
    b_per_w = (B * M) // (nc * ns)
    nch = b_per_w // _W

    @functools.partial(
        pl.kernel, mesh=mesh,
        out_type=jax.ShapeDtypeStruct((B * M, E), _F32),
        scratch_types=[
            pltpu.VMEM((_W,), _I32), pltpu.VMEM((_W,), _I32),
            pltpu.VMEM((_W, E), _F32), pltpu.VMEM((_W, E), _F32),
            pltpu.SemaphoreType.DMA, pltpu.SemaphoreType.DMA,
        ],
    )
    def k(x_hbm, idx_hbm, out_hbm, idx0, idx1, rows0, rows1, sem0, sem1):
        wid = jax.lax.axis_index('s') * nc + jax.lax.axis_index('c')
        base0 = wid * b_per_w

        def issue(c, idx_v, rows_v, sem):
            pltpu.sync_copy(idx_hbm.at[pl.ds(base0 + c * _W, _W)], idx_v)
            pltpu.async_copy(x_hbm.at[idx_v], rows_v, sem)

        def drain(c, idx_v, rows_v, sem):
            pltpu.make_async_copy(x_hbm.at[idx_v], rows_v, sem).wait()
            pltpu.sync_copy(rows_v, out_hbm.at[pl.ds(base0 + c * _W, _W)])

        issue(0, idx0, rows0, sem0)

        @pl.loop(0, nch // 2)
        def _(j):
            c0 = 2 * j
            issue(c0 + 1, idx1, rows1, sem1)
            drain(c0, idx0, rows0, sem0)

            @pl.when(c0 + 2 < nch)
            def _():
                issue(c0 + 2, idx0, rows0, sem0)

            drain(c0 + 1, idx1, rows1, sem1)

    return k(x2_flat, gidx_flat)


def kernel(hidden_phoneme_sequence, sequence_mask, frame_masks, pitch_target,
           energy_target, duration_target, duration_scale, pitch_scale,
           energy_scale,
           dur_c1w, dur_c1b, dur_ln1s, dur_ln1b, dur_c2w, dur_c2b,
           dur_ln2s, dur_ln2b, dur_lw, dur_lb,
           pit_c1w, pit_c1b, pit_ln1s, pit_ln1b, pit_c2w, pit_c2b,
           pit_ln2s, pit_ln2b, pit_lw, pit_lb,
           ene_c1w, ene_c1b, ene_ln1s, ene_ln1b, ene_c2w, ene_c2b,
           ene_ln2s, ene_ln2b, ene_lw, ene_lb,
           pitch_bins, energy_bins, pitch_emb, energy_emb):
    x0 = hidden_phoneme_sequence
    r2 = lambda a: a.reshape(1, -1)

    x0b, x1b, x2, gidx = pl.pallas_call(
        _emb_body,
        grid=(B,),
        in_specs=[
            pl.BlockSpec((1, L, E), lambda b: (b, 0, 0)),
            _const(B, L), _const(B, L), _const(B, L),
            _const(1, NB), _const(1, NB), _const(NB, E), _const(NB, E),
        ],
        out_specs=(pl.BlockSpec((1, L, E), lambda b: (b, 0, 0)),
                   pl.BlockSpec((1, L, E), lambda b: (b, 0, 0)),
                   pl.BlockSpec((1, L, E), lambda b: (b, 0, 0)),
                   pl.BlockSpec((1, 1, M), lambda b: (b, 0, 0))),
        out_shape=(jax.ShapeDtypeStruct((B, L, E), _BF16),
                   jax.ShapeDtypeStruct((B, L, E), _BF16),
                   jax.ShapeDtypeStruct((B, L, E), _F32),
                   jax.ShapeDtypeStruct((B, 1, M), _I32)),
    )(x0, pitch_target, energy_target, duration_target.astype(_I32),
      r2(pitch_bins), r2(energy_bins), pitch_emb, energy_emb)

    xout = _sc_gather(x2.reshape(B * L, E), gidx.reshape(B * M))

    wts = []
    w_specs = []
    for t in ((dur_c1w, dur_c1b, dur_ln1s, dur_ln1b, dur_c2w, dur_c2b,
               dur_ln2s, dur_ln2b, dur_lw, dur_lb),
              (pit_c1w, pit_c1b, pit_ln1s, pit_ln1b, pit_c2w, pit_c2b,
               pit_ln2s, pit_ln2b, pit_lw, pit_lb),
              (ene_c1w, ene_c1b, ene_ln1s, ene_ln1b, ene_c2w, ene_c2b,
               ene_ln2s, ene_ln2b, ene_lw, ene_lb)):
        c1w, c1b, ln1s, ln1b, c2w, c2b, ln2s, ln2b, lw, lb = t
        wts += [c1w, r2(c1b), r2(ln1s), r2(ln1b), c2w, r2(c2b), r2(ln2s),
                r2(ln2b), lw.reshape(1, F), lb.reshape(1, 1)]
        w_specs += [
            _const(K, E, F), _const(1, F), _const(1, F), _const(1, F),
            _const(K, F, F), _const(1, F), _const(1, F), _const(1, F),
            _const(1, F), _const(1, 1),
        ]

    logd, pitch, energy = pl.pallas_call(
        _pred_body,
        grid=(B // _G,),
        in_specs=[pl.BlockSpec((_G, L, E), lambda b: (b, 0, 0)),
                  pl.BlockSpec((_G, L, E), lambda b: (b, 0, 0)),
                  *w_specs],
        out_specs=(pl.BlockSpec((1, 1, _G * L), lambda b: (b, 0, 0)),) * 3,
        out_shape=(jax.ShapeDtypeStruct((B // _G, 1, _G * L), _F32),) * 3,
    )(x0b, x1b, *wts)

    return (logd.reshape(B, L), pitch.reshape(B, L), energy.reshape(B, L),
            xout.reshape(B, M, E), frame_masks)


# speedup vs baseline: 1.2991x; 1.0182x over previous
import functools

import jax
import jax.numpy as jnp
from jax.experimental import pallas as pl
from jax.experimental.pallas import tpu as pltpu
from jax.experimental.pallas import tpu_sc as plsc

B, L, M, E = 16, 512, 2048, 256
F, K, NB = 256, 3, 256
_F32 = jnp.float32
_BF16 = jnp.bfloat16
_I32 = jnp.int32
_W = 128
_G = 2


def _shift_dn(x):
    return jnp.concatenate([jnp.zeros((1, x.shape[1]), x.dtype), x[:-1]], axis=0)


def _shift_up(x):
    return jnp.concatenate([x[1:], jnp.zeros((1, x.shape[1]), x.dtype)], axis=0)


def _conv3(xb, mask_dn, mask_up, w, bias):
    h = jnp.dot(xb, w[1], preferred_element_type=_F32)
    h = h + jnp.dot(_shift_dn(xb) * mask_dn, w[0], preferred_element_type=_F32)
    h = h + jnp.dot(_shift_up(xb) * mask_up, w[2], preferred_element_type=_F32)
    return h + bias[None, :]


def _ln_bf16(h, s, b):
    mu = jnp.mean(h, axis=-1, keepdims=True)
    var = jnp.mean((h - mu) * (h - mu), axis=-1, keepdims=True)
    rs = 1.0 / jnp.sqrt(var + 1e-5)
    return ((h - mu) * rs * s[None, :] + b[None, :]).astype(_BF16)


def _predictor_tail(h1, mask_dn, mask_up, ln1s, ln1b, c2w, c2b,
                    ln2s, ln2b, lw, lb):
    h = _ln_bf16(h1, ln1s, ln1b)
    h = jax.nn.relu(_conv3(h, mask_dn, mask_up, c2w, c2b))
    mu = jnp.mean(h, axis=-1, keepdims=True)
    var = jnp.mean((h - mu) * (h - mu), axis=-1, keepdims=True)
    rs = 1.0 / jnp.sqrt(var + 1e-5)
    hn = ((h - mu) * rs).astype(_BF16)
    w2 = (ln2s * lw)[None, :].astype(_BF16)
    c2 = jnp.sum(ln2b * lw) + lb
    res = jax.lax.dot_general(w2, hn, (((1,), (1,)), ((), ())),
                              preferred_element_type=_F32)
    return res[0] + c2


def _bucket_emb(target, bins, emb_b):
    c = (bins[None, :] < target[:, None]).astype(_BF16)
    ones_col = jnp.ones((L, 1), _BF16)
    c_prev = jnp.concatenate([ones_col, c[:, :-1]], axis=1)
    not_c = jnp.concatenate([1.0 - c[:, :-1], ones_col], axis=1)
    return jnp.dot(c_prev * not_c, emb_b, preferred_element_type=_F32)


def _emb_body(x_ref, pt_ref, et_ref, dur_ref, pbins, ebins, pemb, eemb,
              x0b_ref, x1b_ref, x2_ref, gidx_ref):
    b = pl.program_id(0)
    x0 = x_ref[0]
    x0b_ref[0] = x0.astype(_BF16)
    p_emb = _bucket_emb(pt_ref[b, :], pbins[0], pemb[...].astype(_BF16))
    e_emb = _bucket_emb(et_ref[b, :], ebins[0], eemb[...].astype(_BF16))
    x1 = x0 + p_emb
    x1b_ref[0] = x1.astype(_BF16)
    x2_ref[0] = x1 + e_emb

    df = dur_ref[b, :].astype(_BF16)[None, :]
    tri = (jax.lax.broadcasted_iota(_I32, (L, L), 0)
           < jax.lax.broadcasted_iota(_I32, (L, L), 1)).astype(_BF16)
    excl = jnp.dot(df, tri, preferred_element_type=_F32)
    excl_col = excl.reshape(L, 1)
    miota = jax.lax.broadcasted_iota(_I32, (1, M), 1).astype(_F32)
    cmp_t = (excl_col <= miota).astype(_BF16)
    cnt = jnp.dot(jnp.ones((1, L), _BF16), cmp_t,
                  preferred_element_type=_F32)
    gidx_ref[0, 0, :] = cnt[0].astype(_I32) - 1 + b * L


def _pred_body(x0b_ref, x1b_ref,
               dw1, db1, ds1, dbb1, dw2, db2, ds2, dbb2, dlw, dlb,
               pw1, pb1, ps1, pbb1, pw2, pb2, ps2, pbb2, plw, plb,
               ew1, eb1, es1, ebb1, ew2, eb2, es2, ebb2, elw, elb,
               logd_ref, pitch_ref, energy_ref):
    wb = lambda w: w[...].astype(_BF16)
    x0b = x0b_ref[...].reshape(_G * L, E)
    x1b = x1b_ref[...].reshape(_G * L, E)
    riota = jax.lax.broadcasted_iota(_I32, (_G * L, 1), 0)
    mask_dn = ((riota & (L - 1)) != 0).astype(_BF16)
    mask_up = ((riota & (L - 1)) != (L - 1)).astype(_BF16)
    wdp = jnp.concatenate([wb(dw1), wb(pw1)], axis=2)
    bdp = jnp.concatenate([db1[0], pb1[0]], axis=0)
    hdp = jax.nn.relu(_conv3(x0b, mask_dn, mask_up, wdp, bdp))
    logd_ref[0, 0, :] = _predictor_tail(
        hdp[:, :F], mask_dn, mask_up, ds1[0], dbb1[0], wb(dw2),
        db2[0], ds2[0], dbb2[0], dlw[0], dlb[0, 0])
    pitch_ref[0, 0, :] = _predictor_tail(
        hdp[:, F:], mask_dn, mask_up, ps1[0], pbb1[0], wb(pw2),
        pb2[0], ps2[0], pbb2[0], plw[0], plb[0, 0])
    h1e = jax.nn.relu(_conv3(x1b, mask_dn, mask_up, wb(ew1), eb1[0]))
    energy_ref[0, 0, :] = _predictor_tail(
        h1e, mask_dn, mask_up, es1[0], ebb1[0], wb(ew2),
        eb2[0], es2[0], ebb2[0], elw[0], elb[0, 0])


def _const(*shape):
    nd = len(shape)
    return pl.BlockSpec(shape, lambda b, _n=nd: (0,) * _n)


def _sc_gather(x2_flat, gidx_flat):
    mesh = plsc.VectorSubcoreMesh(core_axis_name='c', subcore_axis_name='s')
    nc, ns = 2, 16
    b_per_w = (B * M) // (nc * ns)
    nch = b_per_w // _W

    @functools.partial(
        pl.kernel, mesh=mesh,
        out_type=jax.ShapeDtypeStruct((B * M, E), _F32),
        scratch_types=[
            pltpu.VMEM((_W,), _I32), pltpu.VMEM((_W,), _I32),
            pltpu.VMEM((_W, E), _F32), pltpu.VMEM((_W, E), _F32),
            pltpu.SemaphoreType.DMA, pltpu.SemaphoreType.DMA,
        ],
    )
    def k(x_hbm, idx_hbm, out_hbm, idx0, idx1, rows0, rows1, sem0, sem1):
        wid = jax.lax.axis_index('s') * nc + jax.lax.axis_index('c')
        base0 = wid * b_per_w

        def issue(c, idx_v, rows_v, sem):
            pltpu.sync_copy(idx_hbm.at[pl.ds(base0 + c * _W, _W)], idx_v)
            pltpu.async_copy(x_hbm.at[idx_v], rows_v, sem)

        def drain(c, idx_v, rows_v, sem):
            pltpu.make_async_copy(x_hbm.at[idx_v], rows_v, sem).wait()
            pltpu.sync_copy(rows_v, out_hbm.at[pl.ds(base0 + c * _W, _W)])

        issue(0, idx0, rows0, sem0)

        @pl.loop(0, nch // 2)
        def _(j):
            c0 = 2 * j
            issue(c0 + 1, idx1, rows1, sem1)
            drain(c0, idx0, rows0, sem0)

            @pl.when(c0 + 2 < nch)
            def _():
                issue(c0 + 2, idx0, rows0, sem0)

            drain(c0 + 1, idx1, rows1, sem1)

    return k(x2_flat, gidx_flat)


def kernel(hidden_phoneme_sequence, sequence_mask, frame_masks, pitch_target,
           energy_target, duration_target, duration_scale, pitch_scale,
           energy_scale,
           dur_c1w, dur_c1b, dur_ln1s, dur_ln1b, dur_c2w, dur_c2b,
           dur_ln2s, dur_ln2b, dur_lw, dur_lb,
           pit_c1w, pit_c1b, pit_ln1s, pit_ln1b, pit_c2w, pit_c2b,
           pit_ln2s, pit_ln2b, pit_lw, pit_lb,
           ene_c1w, ene_c1b, ene_ln1s, ene_ln1b, ene_c2w, ene_c2b,
           ene_ln2s, ene_ln2b, ene_lw, ene_lb,
           pitch_bins, energy_bins, pitch_emb, energy_emb):
    x0 = hidden_phoneme_sequence
    r2 = lambda a: a.reshape(1, -1)

    x0b, x1b, x2, gidx = pl.pallas_call(
        _emb_body,
        grid=(B,),
        in_specs=[
            pl.BlockSpec((1, L, E), lambda b: (b, 0, 0)),
            _const(B, L), _const(B, L), _const(B, L),
            _const(1, NB), _const(1, NB), _const(NB, E), _const(NB, E),
        ],
        out_specs=(pl.BlockSpec((1, L, E), lambda b: (b, 0, 0)),
                   pl.BlockSpec((1, L, E), lambda b: (b, 0, 0)),
                   pl.BlockSpec((1, L, E), lambda b: (b, 0, 0)),
                   pl.BlockSpec((1, 1, M), lambda b: (b, 0, 0))),
        out_shape=(jax.ShapeDtypeStruct((B, L, E), _BF16),
                   jax.ShapeDtypeStruct((B, L, E), _BF16),
                   jax.ShapeDtypeStruct((B, L, E), _F32),
                   jax.ShapeDtypeStruct((B, 1, M), _I32)),
    )(x0, pitch_target, energy_target, duration_target.astype(_I32),
      r2(pitch_bins), r2(energy_bins), pitch_emb, energy_emb)

    xout = _sc_gather(x2.reshape(B * L, E), gidx.reshape(B * M))

    wts = []
    w_specs = []
    for t in ((dur_c1w, dur_c1b, dur_ln1s, dur_ln1b, dur_c2w, dur_c2b,
               dur_ln2s, dur_ln2b, dur_lw, dur_lb),
              (pit_c1w, pit_c1b, pit_ln1s, pit_ln1b, pit_c2w, pit_c2b,
               pit_ln2s, pit_ln2b, pit_lw, pit_lb),
              (ene_c1w, ene_c1b, ene_ln1s, ene_ln1b, ene_c2w, ene_c2b,
               ene_ln2s, ene_ln2b, ene_lw, ene_lb)):
        c1w, c1b, ln1s, ln1b, c2w, c2b, ln2s, ln2b, lw, lb = t
        wts += [c1w, r2(c1b), r2(ln1s), r2(ln1b), c2w, r2(c2b), r2(ln2s),
                r2(ln2b), lw.reshape(1, F), lb.reshape(1, 1)]
        w_specs += [
            _const(K, E, F), _const(1, F), _const(1, F), _const(1, F),
            _const(K, F, F), _const(1, F), _const(1, F), _const(1, F),
            _const(1, F), _const(1, 1),
        ]

    logd, pitch, energy = pl.pallas_call(
        _pred_body,
        grid=(B // _G,),
        in_specs=[pl.BlockSpec((_G, L, E), lambda b: (b, 0, 0)),
                  pl.BlockSpec((_G, L, E), lambda b: (b, 0, 0)),
                  *w_specs],
        out_specs=(pl.BlockSpec((1, 1, _G * L), lambda b: (b, 0, 0)),) * 3,
        out_shape=(jax.ShapeDtypeStruct((B // _G, 1, _G * L), _F32),) * 3,
    )(x0b, x1b, *wts)

    return (logd.reshape(B, L), pitch.reshape(B, L), energy.reshape(B, L),
            xout.reshape(B, M, E), frame_masks)
